# baseline (speedup 1.0000x reference)
import functools

import jax
import jax.numpy as jnp
from jax import lax
from jax.experimental import pallas as pl
from jax.experimental.pallas import tpu as pltpu
from jax.experimental.pallas import tpu_sc as plsc

NC = 2
NS = 16
NW = NC * NS

TB = 128
NBUF = 4
TPV = 4096


def _table_prep(tableT):
    D, V = tableT.shape
    nblk = pl.cdiv(V, TPV)
    H = TPV // 2

    def body(x_ref, y_ref):
        xT = x_ref[...].T
        y_ref[...] = jnp.concatenate([xT[:H], xT[H:]], axis=1)

    return pl.pallas_call(
        body,
        grid=(nblk,),
        in_specs=[pl.BlockSpec((D, TPV), lambda i: (0, i))],
        out_specs=pl.BlockSpec((H, 2 * D), lambda i: (i, 0)),
        out_shape=jax.ShapeDtypeStruct((nblk * H, 2 * D), jnp.float32),
    )(tableT)


def _remap(idx):
    blk = idx // TPV
    j = idx % TPV
    return blk * TPV + 2 * (j % (TPV // 2)) + j // (TPV // 2)


def _out_transpose(x, S, BB, D):
    C8 = D // 8

    def body(x_ref, y_ref):
        xb = x_ref[0]
        yb = jnp.concatenate([xb[:, :D].T, xb[:, D:].T], axis=1)
        for c8 in range(C8):
            y_ref[0, c8, 0] = yb[c8 * 8 : (c8 + 1) * 8, :]

    return pl.pallas_call(
        body,
        grid=(S, BB),
        in_specs=[
            pl.BlockSpec((1, TB // 2, 2 * D), lambda s, bb: (s * BB + bb, 0, 0))
        ],
        out_specs=pl.BlockSpec(
            (1, C8, 1, 8, TB), lambda s, bb: (s, 0, bb, 0, 0)
        ),
        out_shape=jax.ShapeDtypeStruct((S, C8, BB, 8, TB), jnp.float32),
    )(x)


def _make_sc_gather(n_units, V2, D):
    upw = n_units // NW
    mesh = plsc.VectorSubcoreMesh(core_axis_name="c", subcore_axis_name="s")
    N = n_units * TB

    @functools.partial(
        pl.kernel,
        mesh=mesh,
        out_type=jax.ShapeDtypeStruct((N, D), jnp.float32),
        scratch_types=[
            pltpu.VMEM((upw, TB), jnp.int32),
            pltpu.VMEM((NBUF, TB, D), jnp.float32),
        ]
        + [pltpu.SemaphoreType.DMA] * (2 * NBUF),
        compiler_params=pltpu.CompilerParams(use_tc_tiling_on_sc=False),
    )
    def sc_gather(idx_hbm, table_hbm, out_hbm, idx_v, rows_v, *sems):
        gsems = sems[:NBUF]
        wsems = sems[NBUF:]
        wid = lax.axis_index("s") * NC + lax.axis_index("c")
        ubase = wid * upw

        pltpu.sync_copy(idx_hbm.at[pl.ds(ubase, upw)], idx_v)

        def gather_start(ul, b):
            pltpu.make_async_copy(
                table_hbm.at[idx_v.at[ul]], rows_v.at[b], gsems[b]
            ).start()

        def gather_wait(b):
            pltpu.make_async_copy(
                table_hbm.at[idx_v.at[0]], rows_v.at[b], gsems[b]
            ).wait()

        def write_start(ul, b):
            pltpu.make_async_copy(
                rows_v.at[b],
                out_hbm.at[pl.ds((ubase + ul) * TB, TB)],
                wsems[b],
            ).start()

        def write_wait(b):
            pltpu.make_async_copy(
                rows_v.at[b], out_hbm.at[pl.ds(0, TB)], wsems[b]
            ).wait()

        for b in range(NBUF):
            gather_start(b, b)

        def outer(g, carry):
            for b in range(NBUF):
                ul = g * NBUF + b
                gather_wait(b)
                write_start(ul, b)
            for b in range(NBUF):
                jn = (g + 1) * NBUF + b

                @pl.when(jn < upw)
                def _():
                    write_wait(b)
                    gather_start(jn, b)

            return carry

        lax.fori_loop(0, upw // NBUF, outer, 0)

        for b in range(NBUF):
            write_wait(b)

    return sc_gather


def kernel(sequence, table):
    B, S = sequence.shape
    V, D = table.shape
    BB = B // TB
    n_units = S * BB
    assert n_units % NW == 0 and D % 16 == 0

    table2 = _table_prep(table.T)
    V2 = 2 * table2.shape[0]

    idx = (
        _remap(sequence.T)
        .reshape(n_units, 2, TB // 2)
        .transpose(0, 2, 1)
        .reshape(n_units, TB)
    )
    rows = _make_sc_gather(n_units, V2, D)(idx, table2.reshape(V2, D))

    out5d = _out_transpose(rows.reshape(n_units, TB // 2, 2 * D), S, BB, D)

    return out5d.transpose(2, 4, 0, 1, 3).reshape(B, S, D)


# speedup vs baseline: 5.0240x; 5.0240x over previous
import functools

import jax
import jax.numpy as jnp
from jax import lax
from jax.experimental import pallas as pl
from jax.experimental.pallas import tpu as pltpu
from jax.experimental.pallas import tpu_sc as plsc

NC = 2
NS = 16
NW = NC * NS

TB = 128
NBUF = 4
TPV = 4096


def _table_prep(tableT):
    D, V = tableT.shape
    nblk = pl.cdiv(V, TPV)
    H = TPV // 2

    def body(x_ref, y_ref):
        xT = x_ref[...].T
        y_ref[...] = jnp.concatenate([xT[:H], xT[H:]], axis=1)

    return pl.pallas_call(
        body,
        grid=(nblk,),
        in_specs=[pl.BlockSpec((D, TPV), lambda i: (0, i))],
        out_specs=pl.BlockSpec((H, 2 * D), lambda i: (i, 0)),
        out_shape=jax.ShapeDtypeStruct((nblk * H, 2 * D), jnp.float32),
    )(tableT)


def _remap(idx):
    blk = idx // TPV
    j = idx % TPV
    return blk * TPV + 2 * (j % (TPV // 2)) + j // (TPV // 2)


def _out_transpose(x, S, BB, D):
    C8 = D // 8

    def body(x_ref, y_ref):
        xs = x_ref[...]
        ya = jnp.transpose(xs[:, :, :D], (0, 2, 1))
        yo = jnp.transpose(xs[:, :, D:], (0, 2, 1))
        y = jnp.concatenate([ya, yo], axis=2)
        y4 = y.reshape(BB, C8, 8, TB).transpose(1, 0, 2, 3)
        y_ref[0] = y4

    return pl.pallas_call(
        body,
        grid=(S,),
        in_specs=[pl.BlockSpec((BB, TB // 2, 2 * D), lambda s: (s, 0, 0))],
        out_specs=pl.BlockSpec(
            (1, C8, BB, 8, TB), lambda s: (s, 0, 0, 0, 0)
        ),
        out_shape=jax.ShapeDtypeStruct((S, C8, BB, 8, TB), jnp.float32),
    )(x)


def _make_sc_gather(n_units, V2, D):
    upw = n_units // NW
    mesh = plsc.VectorSubcoreMesh(core_axis_name="c", subcore_axis_name="s")
    N = n_units * TB

    @functools.partial(
        pl.kernel,
        mesh=mesh,
        out_type=jax.ShapeDtypeStruct((N, D), jnp.float32),
        scratch_types=[
            pltpu.VMEM((upw, TB), jnp.int32),
            pltpu.VMEM((NBUF, TB, D), jnp.float32),
        ]
        + [pltpu.SemaphoreType.DMA] * (2 * NBUF),
        compiler_params=pltpu.CompilerParams(use_tc_tiling_on_sc=False),
    )
    def sc_gather(idx_hbm, table_hbm, out_hbm, idx_v, rows_v, *sems):
        gsems = sems[:NBUF]
        wsems = sems[NBUF:]
        wid = lax.axis_index("s") * NC + lax.axis_index("c")
        ubase = wid * upw

        pltpu.sync_copy(idx_hbm.at[pl.ds(ubase, upw)], idx_v)

        def gather_start(ul, b):
            pltpu.make_async_copy(
                table_hbm.at[idx_v.at[ul]], rows_v.at[b], gsems[b]
            ).start()

        def gather_wait(b):
            pltpu.make_async_copy(
                table_hbm.at[idx_v.at[0]], rows_v.at[b], gsems[b]
            ).wait()

        def write_start(ul, b):
            pltpu.make_async_copy(
                rows_v.at[b],
                out_hbm.at[pl.ds((ubase + ul) * TB, TB)],
                wsems[b],
            ).start()

        def write_wait(b):
            pltpu.make_async_copy(
                rows_v.at[b], out_hbm.at[pl.ds(0, TB)], wsems[b]
            ).wait()

        for b in range(NBUF):
            gather_start(b, b)

        def outer(g, carry):
            for b in range(NBUF):
                ul = g * NBUF + b
                gather_wait(b)
                write_start(ul, b)
            for b in range(NBUF):
                jn = (g + 1) * NBUF + b

                @pl.when(jn < upw)
                def _():
                    write_wait(b)
                    gather_start(jn, b)

            return carry

        lax.fori_loop(0, upw // NBUF, outer, 0)

        for b in range(NBUF):
            write_wait(b)

    return sc_gather


def kernel(sequence, table):
    B, S = sequence.shape
    V, D = table.shape
    BB = B // TB
    n_units = S * BB
    assert n_units % NW == 0 and D % 16 == 0

    table2 = _table_prep(table.T)
    V2 = 2 * table2.shape[0]

    idx = (
        _remap(sequence.T)
        .reshape(n_units, 2, TB // 2)
        .transpose(0, 2, 1)
        .reshape(n_units, TB)
    )
    rows = _make_sc_gather(n_units, V2, D)(idx, table2.reshape(V2, D))

    out5d = _out_transpose(rows.reshape(n_units, TB // 2, 2 * D), S, BB, D)

    return out5d.transpose(2, 4, 0, 1, 3).reshape(B, S, D)
